# baseline (speedup 1.0000x reference)
import functools

import jax
import jax.numpy as jnp
from jax import lax
from jax.experimental import pallas as pl
from jax.experimental.pallas import tpu as pltpu
from jax.experimental.pallas import tpu_sc as plsc

D = 32
B = 16384 * 50

_info = plsc.get_sparse_core_info()
_NC, _NS = _info.num_cores, _info.num_subcores
NW = _NC * _NS
PER_W = B // NW
CHUNK = 1600
NCH = PER_W // CHUNK

_mesh = plsc.VectorSubcoreMesh(core_axis_name="c", subcore_axis_name="s")


@functools.partial(
    pl.kernel,
    mesh=_mesh,
    out_type=jax.ShapeDtypeStruct((B, D), jnp.float32),
    scratch_types=[
        pltpu.VMEM((PER_W,), jnp.int32),
        pltpu.VMEM((CHUNK, D), jnp.float32),
        pltpu.VMEM((CHUNK, D), jnp.float32),
        pltpu.SemaphoreType.DMA,
        pltpu.SemaphoreType.DMA,
        pltpu.SemaphoreType.DMA,
        pltpu.SemaphoreType.DMA,
    ],
    compiler_params=pltpu.CompilerParams(use_tc_tiling_on_sc=False),
)
def _gather_kernel(idx_hbm, table_hbm, out_hbm, idx_v, buf0, buf1, g0, g1, s0, s1):
    wid = lax.axis_index("s") * _NC + lax.axis_index("c")
    base = wid * PER_W
    pltpu.sync_copy(idx_hbm.at[pl.ds(base, PER_W)], idx_v)
    bufs, gsems, ssems = (buf0, buf1), (g0, g1), (s0, s1)

    def start_gather(g):
        b = g % 2
        return pltpu.async_copy(
            table_hbm.at[idx_v.at[pl.ds(g * CHUNK, CHUNK)]], bufs[b], gsems[b]
        )

    gathers = [None] * NCH
    stores = [None] * NCH
    gathers[0] = start_gather(0)
    for g in range(NCH):
        b = g % 2
        if g + 1 < NCH:
            if g - 1 >= 0:
                stores[g - 1].wait()
            gathers[g + 1] = start_gather(g + 1)
        gathers[g].wait()
        stores[g] = pltpu.async_copy(
            bufs[b], out_hbm.at[pl.ds(base + g * CHUNK, CHUNK)], ssems[b]
        )
    stores[NCH - 2].wait()
    stores[NCH - 1].wait()


def kernel(inputs, table):
    flat = inputs.reshape(-1)
    out = _gather_kernel(flat, table)
    return out.reshape(inputs.shape + (table.shape[1],))


# speedup vs baseline: 2.0032x; 2.0032x over previous
import functools

import jax
import jax.numpy as jnp
from jax import lax
from jax.experimental import pallas as pl
from jax.experimental.pallas import tpu as pltpu
from jax.experimental.pallas import tpu_sc as plsc

ROWS = 16384
HIST = 50
D = 32

_info = plsc.get_sparse_core_info()
_NC, _NS = _info.num_cores, _info.num_subcores
NW = _NC * _NS
ROWS_W = ROWS // NW
CHUNK = 32
NCH = ROWS_W // CHUNK

_mesh = plsc.VectorSubcoreMesh(core_axis_name="c", subcore_axis_name="s")


@functools.partial(
    pl.kernel,
    mesh=_mesh,
    out_type=jax.ShapeDtypeStruct((ROWS, HIST, D), jnp.float32),
    scratch_types=[
        pltpu.VMEM((ROWS_W, HIST), jnp.int32),
        pltpu.VMEM((CHUNK, HIST, D), jnp.float32),
        pltpu.VMEM((CHUNK, HIST, D), jnp.float32),
        pltpu.SemaphoreType.DMA,
        pltpu.SemaphoreType.DMA,
        pltpu.SemaphoreType.DMA,
        pltpu.SemaphoreType.DMA,
    ],
    compiler_params=pltpu.CompilerParams(use_tc_tiling_on_sc=False),
)
def _gather_kernel(idx_hbm, table_hbm, out_hbm, idx_v, buf0, buf1, g0, g1, s0, s1):
    wid = lax.axis_index("s") * _NC + lax.axis_index("c")
    base = wid * ROWS_W
    pltpu.sync_copy(idx_hbm.at[pl.ds(base, ROWS_W)], idx_v)

    def do_chunk(g, buf, gsem, ssem):
        row0 = g * CHUNK
        cps = [
            pltpu.async_copy(table_hbm.at[idx_v.at[row0 + r]], buf.at[r], gsem)
            for r in range(CHUNK)
        ]
        for c in cps:
            c.wait()
        return pltpu.async_copy(
            buf, out_hbm.at[pl.ds(base + row0, CHUNK)], ssem
        )

    def body(o, carry):
        st0 = do_chunk(2 * o, buf0, g0, s0)
        st1 = do_chunk(2 * o + 1, buf1, g1, s1)
        st0.wait()
        st1.wait()
        return carry

    lax.fori_loop(0, NCH // 2, body, 0)


def kernel(inputs, table):
    return _gather_kernel(inputs, table)
